# Initial kernel scaffold; baseline (speedup 1.0000x reference)
import jax
import jax.numpy as jnp
from jax.experimental import pallas as pl


def kernel(h, weight, bias, mean_scale):
    raise NotImplementedError("write your pallas kernel here")


# speedup vs baseline: 10.4858x; 10.4858x over previous
import functools

import jax
import jax.numpy as jnp
from jax.experimental import pallas as pl

_NODES = 4096
_COL_TILE = 128


def _graphnorm_block(h_ref, w_ref, b_ref, ms_ref, out_ref):
    x = h_ref[:, :]
    mean = jnp.mean(x, axis=0, keepdims=True)
    centered = x - mean * ms_ref[:, :]
    var = jnp.mean(centered * centered, axis=0, keepdims=True)
    inv = jax.lax.rsqrt(var + 1e-6)
    out_ref[:, :] = (w_ref[:, :] * inv) * centered + b_ref[:, :]


@jax.jit
def kernel(h, weight, bias, mean_scale):
    rows, hidden = h.shape
    batch = rows // _NODES
    cw = _COL_TILE
    w2 = weight.reshape(1, hidden)
    b2 = bias.reshape(1, hidden)
    ms2 = mean_scale.reshape(1, hidden)
    grid = (batch, hidden // cw)
    out = pl.pallas_call(
        _graphnorm_block,
        grid=grid,
        in_specs=[
            pl.BlockSpec((_NODES, cw), lambda i, j: (i, j)),
            pl.BlockSpec((1, cw), lambda i, j: (0, j)),
            pl.BlockSpec((1, cw), lambda i, j: (0, j)),
            pl.BlockSpec((1, cw), lambda i, j: (0, j)),
        ],
        out_specs=pl.BlockSpec((_NODES, cw), lambda i, j: (i, j)),
        out_shape=jax.ShapeDtypeStruct((rows, hidden), h.dtype),
    )(h, w2, b2, ms2)
    return out


# speedup vs baseline: 12.0455x; 1.1487x over previous
import functools

import jax
import jax.numpy as jnp
from jax.experimental import pallas as pl

_NODES = 4096
_COL_TILE = 128


def _graphnorm_block(h_ref, w_ref, b_ref, ms_ref, out_ref):
    x = h_ref[:, :]
    m1 = jnp.mean(x, axis=0, keepdims=True)
    m2 = jnp.mean(x * x, axis=0, keepdims=True)
    s = ms_ref[:, :]
    var = m2 - (2.0 * s - s * s) * (m1 * m1)
    inv = jax.lax.rsqrt(var + 1e-6)
    a = w_ref[:, :] * inv
    out_ref[:, :] = a * x + (b_ref[:, :] - a * (s * m1))


@jax.jit
def kernel(h, weight, bias, mean_scale):
    rows, hidden = h.shape
    batch = rows // _NODES
    cw = _COL_TILE
    w2 = weight.reshape(1, hidden)
    b2 = bias.reshape(1, hidden)
    ms2 = mean_scale.reshape(1, hidden)
    grid = (batch, hidden // cw)
    out = pl.pallas_call(
        _graphnorm_block,
        grid=grid,
        in_specs=[
            pl.BlockSpec((_NODES, cw), lambda i, j: (i, j)),
            pl.BlockSpec((1, cw), lambda i, j: (0, j)),
            pl.BlockSpec((1, cw), lambda i, j: (0, j)),
            pl.BlockSpec((1, cw), lambda i, j: (0, j)),
        ],
        out_specs=pl.BlockSpec((_NODES, cw), lambda i, j: (i, j)),
        out_shape=jax.ShapeDtypeStruct((rows, hidden), h.dtype),
    )(h, w2, b2, ms2)
    return out


# speedup vs baseline: 14.9677x; 1.2426x over previous
import functools

import jax
import jax.numpy as jnp
from jax.experimental import pallas as pl

_NODES = 4096
_COL_TILE = 256


def _graphnorm_block(h_ref, w_ref, b_ref, ms_ref, out_ref):
    x = h_ref[:, :]
    m1 = jnp.mean(x, axis=0, keepdims=True)
    m2 = jnp.mean(x * x, axis=0, keepdims=True)
    s = ms_ref[:, :]
    var = m2 - (2.0 * s - s * s) * (m1 * m1)
    inv = jax.lax.rsqrt(var + 1e-6)
    a = w_ref[:, :] * inv
    out_ref[:, :] = a * x + (b_ref[:, :] - a * (s * m1))


@jax.jit
def kernel(h, weight, bias, mean_scale):
    rows, hidden = h.shape
    batch = rows // _NODES
    cw = _COL_TILE
    w2 = weight.reshape(1, hidden)
    b2 = bias.reshape(1, hidden)
    ms2 = mean_scale.reshape(1, hidden)
    grid = (batch, hidden // cw)
    out = pl.pallas_call(
        _graphnorm_block,
        grid=grid,
        in_specs=[
            pl.BlockSpec((_NODES, cw), lambda i, j: (i, j)),
            pl.BlockSpec((1, cw), lambda i, j: (0, j)),
            pl.BlockSpec((1, cw), lambda i, j: (0, j)),
            pl.BlockSpec((1, cw), lambda i, j: (0, j)),
        ],
        out_specs=pl.BlockSpec((_NODES, cw), lambda i, j: (i, j)),
        out_shape=jax.ShapeDtypeStruct((rows, hidden), h.dtype),
    )(h, w2, b2, ms2)
    return out


# speedup vs baseline: 15.4622x; 1.0330x over previous
import functools

import jax
import jax.numpy as jnp
from jax.experimental import pallas as pl

_NODES = 4096
_COL_TILE = 512


def _graphnorm_block(h_ref, w_ref, b_ref, ms_ref, out_ref):
    x = h_ref[:, :]
    m1 = jnp.mean(x, axis=0, keepdims=True)
    m2 = jnp.mean(x * x, axis=0, keepdims=True)
    s = ms_ref[:, :]
    var = m2 - (2.0 * s - s * s) * (m1 * m1)
    inv = jax.lax.rsqrt(var + 1e-6)
    a = w_ref[:, :] * inv
    out_ref[:, :] = a * x + (b_ref[:, :] - a * (s * m1))


@jax.jit
def kernel(h, weight, bias, mean_scale):
    rows, hidden = h.shape
    batch = rows // _NODES
    cw = _COL_TILE
    w2 = weight.reshape(1, hidden)
    b2 = bias.reshape(1, hidden)
    ms2 = mean_scale.reshape(1, hidden)
    grid = (batch, hidden // cw)
    out = pl.pallas_call(
        _graphnorm_block,
        grid=grid,
        in_specs=[
            pl.BlockSpec((_NODES, cw), lambda i, j: (i, j)),
            pl.BlockSpec((1, cw), lambda i, j: (0, j)),
            pl.BlockSpec((1, cw), lambda i, j: (0, j)),
            pl.BlockSpec((1, cw), lambda i, j: (0, j)),
        ],
        out_specs=pl.BlockSpec((_NODES, cw), lambda i, j: (i, j)),
        out_shape=jax.ShapeDtypeStruct((rows, hidden), h.dtype),
    )(h, w2, b2, ms2)
    return out


# speedup vs baseline: 15.4704x; 1.0005x over previous
import functools

import jax
import jax.numpy as jnp
from jax.experimental import pallas as pl

_NODES = 4096
_COL_TILE = 512


_GRAPHS_PER_BLOCK = 1


def _graphnorm_block(h_ref, w_ref, b_ref, ms_ref, out_ref):
    s = ms_ref[:, :]
    for g in range(_GRAPHS_PER_BLOCK):
        x = h_ref[pl.ds(g * _NODES, _NODES), :]
        m1 = jnp.mean(x, axis=0, keepdims=True)
        m2 = jnp.mean(x * x, axis=0, keepdims=True)
        var = m2 - (2.0 * s - s * s) * (m1 * m1)
        inv = jax.lax.rsqrt(var + 1e-6)
        a = w_ref[:, :] * inv
        out_ref[pl.ds(g * _NODES, _NODES), :] = a * x + (b_ref[:, :] - a * (s * m1))


@jax.jit
def kernel(h, weight, bias, mean_scale):
    rows, hidden = h.shape
    batch = rows // _NODES
    cw = _COL_TILE
    w2 = weight.reshape(1, hidden)
    b2 = bias.reshape(1, hidden)
    ms2 = mean_scale.reshape(1, hidden)
    grid = (batch // _GRAPHS_PER_BLOCK, hidden // cw)
    rows_pb = _NODES * _GRAPHS_PER_BLOCK
    out = pl.pallas_call(
        _graphnorm_block,
        grid=grid,
        in_specs=[
            pl.BlockSpec((rows_pb, cw), lambda i, j: (i, j)),
            pl.BlockSpec((1, cw), lambda i, j: (0, j)),
            pl.BlockSpec((1, cw), lambda i, j: (0, j)),
            pl.BlockSpec((1, cw), lambda i, j: (0, j)),
        ],
        out_specs=pl.BlockSpec((rows_pb, cw), lambda i, j: (i, j)),
        out_shape=jax.ShapeDtypeStruct((rows, hidden), h.dtype),
    )(h, w2, b2, ms2)
    return out
